# Initial kernel scaffold; baseline (speedup 1.0000x reference)
import jax
import jax.numpy as jnp
from jax.experimental import pallas as pl


def kernel(xyz_low, xyz_high, feat_low, feat_high, W, b, gamma, beta):
    raise NotImplementedError("write your pallas kernel here")


# speedup vs baseline: 19.7075x; 19.7075x over previous
import functools

import jax
import jax.numpy as jnp
from jax.experimental import pallas as pl

B, NL, NH, CL, CH, OUT = 8, 4096, 1024, 128, 256, 256

TQ = 256
TN = 512


def _main_body(qT_ref, hT_ref, fhT_ref, fl_ref, w1_ref, w2_ref, b_ref,
               y_ref, stats_ref):
    q = qT_ref[0]
    h = hT_ref[0]
    xy = jax.lax.dot_general(q, h, (((0,), (0,)), ((), ())),
                             preferred_element_type=jnp.float32)
    x2 = jnp.sum(q * q, axis=0)
    y2 = jnp.sum(h * h, axis=0)
    d2 = x2[:, None] + y2[None, :] - 2.0 * xy

    iota = jax.lax.broadcasted_iota(jnp.int32, (TQ, NH), 1)
    big = jnp.float32(jnp.inf)

    def min3(d):
        m = jnp.min(d, axis=1)
        cand = jnp.where(d == m[:, None], iota, NH)
        i = jnp.min(cand, axis=1)
        d_next = jnp.where(iota == i[:, None], big, d)
        return m, i, d_next

    m1, i1, d2b = min3(d2)
    m2, i2, d2c = min3(d2b)
    m3, i3, _ = min3(d2c)

    def invd(m):
        d = jnp.sqrt(jnp.maximum(m, 0.0))
        return 1.0 / jnp.maximum(d, 1e-8)

    w1 = invd(m1)
    w2 = invd(m2)
    w3 = invd(m3)
    s = w1 + w2 + w3
    w1, w2, w3 = w1 / s, w2 / s, w3 / s

    S = (jnp.where(iota == i1[:, None], w1[:, None], 0.0)
         + jnp.where(iota == i2[:, None], w2[:, None], 0.0)
         + jnp.where(iota == i3[:, None], w3[:, None], 0.0))
    f_interp = jax.lax.dot_general(S, fhT_ref[0], (((1,), (0,)), ((), ())),
                                   preferred_element_type=jnp.float32)

    y = jax.lax.dot_general(w1_ref[...], f_interp, (((1,), (1,)), ((), ())),
                            preferred_element_type=jnp.float32)
    y += jax.lax.dot_general(w2_ref[...], fl_ref[0], (((1,), (0,)), ((), ())),
                             preferred_element_type=jnp.float32)
    y += b_ref[...]
    y_ref[0] = y

    ps = jnp.sum(y, axis=1)
    pss = jnp.sum(y * y, axis=1)
    part = jnp.stack([ps, pss], axis=0)
    first = jnp.logical_and(pl.program_id(0) == 0, pl.program_id(1) == 0)

    @pl.when(first)
    def _():
        stats_ref[...] = part

    @pl.when(jnp.logical_not(first))
    def _():
        stats_ref[...] += part


def _norm_body(y_ref, sc_ref, sh_ref, o_ref):
    o_ref[0] = jnp.maximum(y_ref[0] * sc_ref[...] + sh_ref[...], 0.0)


@jax.jit
def kernel(xyz_low, xyz_high, feat_low, feat_high, W, b, gamma, beta):
    qT = jnp.transpose(xyz_low, (0, 2, 1))
    hT = jnp.transpose(xyz_high, (0, 2, 1))
    fhT = jnp.transpose(feat_high, (0, 2, 1))
    W1 = W[:, :CH]
    W2 = W[:, CH:]
    bb = b[:, None]

    grid = (B, NL // TQ)
    y, stats = pl.pallas_call(
        _main_body,
        grid=grid,
        in_specs=[
            pl.BlockSpec((1, 3, TQ), lambda bi, i: (bi, 0, i)),
            pl.BlockSpec((1, 3, NH), lambda bi, i: (bi, 0, 0)),
            pl.BlockSpec((1, NH, CH), lambda bi, i: (bi, 0, 0)),
            pl.BlockSpec((1, CL, TQ), lambda bi, i: (bi, 0, i)),
            pl.BlockSpec((OUT, CH), lambda bi, i: (0, 0)),
            pl.BlockSpec((OUT, CL), lambda bi, i: (0, 0)),
            pl.BlockSpec((OUT, 1), lambda bi, i: (0, 0)),
        ],
        out_specs=[
            pl.BlockSpec((1, OUT, TQ), lambda bi, i: (bi, 0, i)),
            pl.BlockSpec((2, OUT), lambda bi, i: (0, 0)),
        ],
        out_shape=[
            jax.ShapeDtypeStruct((B, OUT, NL), jnp.float32),
            jax.ShapeDtypeStruct((2, OUT), jnp.float32),
        ],
    )(qT, hT, fhT, feat_low, W1, W2, bb)

    n = jnp.float32(B * NL)
    mean = stats[0] / n
    var = jnp.maximum(stats[1] / n - mean * mean, 0.0)
    scale = gamma / jnp.sqrt(var + 1e-5)
    shift = beta - mean * scale

    out = pl.pallas_call(
        _norm_body,
        grid=(B, NL // TN),
        in_specs=[
            pl.BlockSpec((1, OUT, TN), lambda bi, i: (bi, 0, i)),
            pl.BlockSpec((OUT, 1), lambda bi, i: (0, 0)),
            pl.BlockSpec((OUT, 1), lambda bi, i: (0, 0)),
        ],
        out_specs=pl.BlockSpec((1, OUT, TN), lambda bi, i: (bi, 0, i)),
        out_shape=jax.ShapeDtypeStruct((B, OUT, NL), jnp.float32),
    )(y, scale[:, None], shift[:, None])
    return out
